# baseline (speedup 1.0000x reference)
import functools

import jax
import jax.numpy as jnp
from jax import lax
from jax.experimental import pallas as pl
from jax.experimental.pallas import tpu as pltpu
from jax.experimental.pallas import tpu_sc as plsc

N_FACTORS = 32
BATCH = 16384

_NC = 2
_NS = 16
_NW = _NC * _NS
_BW = BATCH // _NW
_GCHUNK = 128
_NG = _BW // _GCHUNK
_ROWS_PER_STEP = 16


def _mf_body(uid_hbm, iid_hbm, uemb_hbm, iemb_hbm, ubias_hbm, ibias_hbm,
             gbias_hbm, out_hbm,
             uidx_v, iidx_v, urows_v, irows_v, ub_v, ib_v, gb_v, out_v, sem):
    wid = lax.axis_index("s") * _NC + lax.axis_index("c")
    base = wid * _BW

    pltpu.sync_copy(uid_hbm.at[pl.ds(base, _BW)], uidx_v)
    pltpu.sync_copy(iid_hbm.at[pl.ds(base, _BW)], iidx_v)
    pltpu.sync_copy(gbias_hbm, gb_v)

    copies = []
    for g in range(_NG):
        s = pl.ds(g * _GCHUNK, _GCHUNK)
        copies.append(pltpu.async_copy(uemb_hbm.at[uidx_v.at[s]], urows_v.at[s], sem))
        copies.append(pltpu.async_copy(iemb_hbm.at[iidx_v.at[s]], irows_v.at[s], sem))
        copies.append(pltpu.async_copy(ubias_hbm.at[uidx_v.at[s]], ub_v.at[s], sem))
        copies.append(pltpu.async_copy(ibias_hbm.at[iidx_v.at[s]], ib_v.at[s], sem))
    for cp in copies:
        cp.wait()

    lane = lax.iota(jnp.int32, 16)
    gb = gb_v[...]

    def step(c, carry):
        row0 = c * _ROWS_PER_STEP
        ridx = row0 + lane
        acc = jnp.zeros((16,), jnp.float32)
        for f in range(N_FACTORS):
            fidx = jnp.full((16,), f, jnp.int32)
            ug = plsc.load_gather(urows_v, [ridx, fidx])
            ig = plsc.load_gather(irows_v, [ridx, fidx])
            acc = acc + ug * ig
        pred = acc + ub_v[pl.ds(row0, 16)] + ib_v[pl.ds(row0, 16)] + gb
        pred = jnp.minimum(jnp.maximum(pred, 1.0), 5.0)
        out_v[pl.ds(row0, 16)] = pred
        return carry

    lax.fori_loop(0, _BW // _ROWS_PER_STEP, step, 0)
    pltpu.sync_copy(out_v, out_hbm.at[pl.ds(base, _BW)])


@jax.jit
def kernel(user_ids, item_ids, user_emb, item_emb, user_bias, item_bias,
           global_bias):
    mesh = plsc.VectorSubcoreMesh(core_axis_name="c", subcore_axis_name="s")
    run = pl.kernel(
        _mf_body,
        mesh=mesh,
        compiler_params=pltpu.CompilerParams(
            needs_layout_passes=False, use_tc_tiling_on_sc=False),
        out_type=jax.ShapeDtypeStruct((BATCH,), jnp.float32),
        scratch_types=[
            pltpu.VMEM((_BW,), jnp.int32),
            pltpu.VMEM((_BW,), jnp.int32),
            pltpu.VMEM((_BW, N_FACTORS), jnp.float32),
            pltpu.VMEM((_BW, N_FACTORS), jnp.float32),
            pltpu.VMEM((_BW,), jnp.float32),
            pltpu.VMEM((_BW,), jnp.float32),
            pltpu.VMEM((16,), jnp.float32),
            pltpu.VMEM((_BW,), jnp.float32),
            pltpu.SemaphoreType.DMA,
        ],
    )
    return run(user_ids, item_ids, user_emb, item_emb,
               user_bias.reshape(-1), item_bias.reshape(-1),
               jnp.broadcast_to(global_bias, (16,)))
